# baseline (speedup 1.0000x reference)
import functools

import jax
import jax.numpy as jnp
from jax import lax
from jax.experimental import pallas as pl
from jax.experimental.pallas import tpu as pltpu
from jax.experimental.pallas import tpu_sc as plsc

NUM_BUCKETS = 1000000
PROJ_DIM = 64
MODEL_DIM = 768
LANES = 16

NUM_CORES = 2
NUM_SUBCORES = 16
D_PER_CORE = PROJ_DIM // NUM_CORES

B_TOTAL = 4 * 8192
TOK_PER_TEC = B_TOTAL // NUM_SUBCORES
GCHUNK = 128
N_GC = TOK_PER_TEC // GCHUNK

HASH_PIECE = 512
N_HP = TOK_PER_TEC // HASH_PIECE
IDS_BUF = HASH_PIECE + 8

SEG = 62464
TAIL_OFF = NUM_SUBCORES * SEG
TAIL = 512
TAIL2_OFF = TAIL_OFF + TAIL
TAIL2 = 128
ROW_BUF = TAIL2_OFF + TAIL2


def _sc_body(idse_hbm, tt_hbm, tailt_hbm, embt_hbm,
             row_sh0, row_sh1, ids_v, idx_v, vals_v, sem_f, sem_g):
    cid = lax.axis_index("c")
    sid = lax.axis_index("s")
    d0 = cid * D_PER_CORE
    base = sid * TOK_PER_TEC

    for piece in range(N_HP):
        pltpu.sync_copy(
            idse_hbm.at[pl.ds(base + piece * HASH_PIECE, IDS_BUF)], ids_v)

        def _hash_step(_, carry):
            off, jrow, icol = carry
            p = ids_v[pl.ds(off + 7, LANES)]
            a = ids_v[pl.ds(off + 8, LANES)]
            t = ((p * 1234) % 1000) * 1000 + p * 567 + a
            idx_v[jrow, pl.ds(icol, LANES)] = t % NUM_BUCKETS
            nicol = icol + LANES
            wrap = nicol == GCHUNK
            return (off + LANES,
                    jnp.where(wrap, jrow + 1, jrow),
                    jnp.where(wrap, jnp.int32(0), nicol))

        lax.fori_loop(
            jnp.int32(0), jnp.int32(HASH_PIECE // LANES), _hash_step,
            (jnp.int32(0), jnp.int32(piece * (HASH_PIECE // GCHUNK)),
             jnp.int32(0)))

        if piece == 0:
            @pl.when(jnp.logical_and(sid != jnp.int32(0),
                                     sid % jnp.int32(4) == jnp.int32(0)))
            def _fix_row_start():
                v = idx_v[jnp.int32(0), pl.ds(0, LANES)]
                a0 = ids_v[pl.ds(8, LANES)]
                lane = lax.iota(jnp.int32, LANES)
                idx_v[jnp.int32(0), pl.ds(0, LANES)] = jnp.where(
                    lane == 0, a0 % NUM_BUCKETS, v)

    soff = pl.multiple_of(sid * jnp.int32(SEG), 128)
    pltpu.sync_copy(tt_hbm.at[d0].at[pl.ds(soff, SEG)],
                    row_sh0.at[pl.ds(soff, SEG)])

    @pl.when(sid == jnp.int32(NUM_SUBCORES - 1))
    def _prime_tail():
        pltpu.sync_copy(tt_hbm.at[d0].at[pl.ds(TAIL_OFF, TAIL)],
                        row_sh0.at[pl.ds(TAIL_OFF, TAIL)])
        pltpu.sync_copy(tailt_hbm.at[pl.ds(d0 * TAIL2, TAIL2)],
                        row_sh0.at[pl.ds(TAIL2_OFF, TAIL2)])

    plsc.subcore_barrier()

    bufs = (row_sh0, row_sh1)
    for d in range(D_PER_CORE):
        cur = bufs[d % 2]
        nxt = bufs[(d + 1) % 2]
        if d + 1 < D_PER_CORE:
            pltpu.async_copy(
                tt_hbm.at[d0 + jnp.int32(d + 1)].at[pl.ds(soff, SEG)],
                nxt.at[pl.ds(soff, SEG)], sem_f)

            @pl.when(sid == jnp.int32(NUM_SUBCORES - 1))
            def _fill_tail(nxt=nxt, d=d):
                pltpu.async_copy(
                    tt_hbm.at[d0 + jnp.int32(d + 1)].at[
                        pl.ds(TAIL_OFF, TAIL)],
                    nxt.at[pl.ds(TAIL_OFF, TAIL)], sem_f)
                pltpu.async_copy(
                    tailt_hbm.at[pl.ds((d0 + jnp.int32(d + 1)) * TAIL2,
                                       TAIL2)],
                    nxt.at[pl.ds(TAIL2_OFF, TAIL2)], sem_f)

        copies = [
            pltpu.async_copy(cur.at[idx_v.at[jnp.int32(g)]],
                             vals_v.at[pl.ds(g * GCHUNK, GCHUNK)], sem_g)
            for g in range(N_GC)
        ]
        for c in copies:
            c.wait()
        pltpu.sync_copy(
            vals_v,
            embt_hbm.at[d0 + jnp.int32(d)].at[pl.ds(base, TOK_PER_TEC)])

        if d + 1 < D_PER_CORE:
            pltpu.make_async_copy(
                tt_hbm.at[d0 + jnp.int32(d + 1)].at[pl.ds(soff, SEG)],
                nxt.at[pl.ds(soff, SEG)], sem_f).wait()

            @pl.when(sid == jnp.int32(NUM_SUBCORES - 1))
            def _drain_tail(nxt=nxt, d=d):
                pltpu.make_async_copy(
                    tt_hbm.at[d0 + jnp.int32(d + 1)].at[
                        pl.ds(TAIL_OFF, TAIL)],
                    nxt.at[pl.ds(TAIL_OFF, TAIL)], sem_f).wait()
                pltpu.make_async_copy(
                    tailt_hbm.at[pl.ds((d0 + jnp.int32(d + 1)) * TAIL2,
                                       TAIL2)],
                    nxt.at[pl.ds(TAIL2_OFF, TAIL2)], sem_f).wait()

        plsc.subcore_barrier()


_sc_gather = functools.partial(
    pl.kernel,
    out_type=jax.ShapeDtypeStruct((PROJ_DIM, B_TOTAL), jnp.float32),
    mesh=plsc.VectorSubcoreMesh(core_axis_name="c", subcore_axis_name="s"),
    scratch_types=[
        pltpu.VMEM_SHARED((ROW_BUF,), jnp.float32),
        pltpu.VMEM_SHARED((ROW_BUF,), jnp.float32),
        pltpu.VMEM((IDS_BUF,), jnp.int32),
        pltpu.VMEM((N_GC, GCHUNK), jnp.int32),
        pltpu.VMEM((TOK_PER_TEC,), jnp.float32),
        pltpu.SemaphoreType.DMA,
        pltpu.SemaphoreType.DMA,
    ],
)(_sc_body)


def _mm_body(embt_ref, wt_ref, out_ref):
    out_ref[...] = lax.dot_general(
        embt_ref[...], wt_ref[...],
        dimension_numbers=(((0,), (0,)), ((), ())),
        preferred_element_type=jnp.float32)


_MM_ROWS = 2048


def _project(embt, wt):
    return pl.pallas_call(
        _mm_body,
        grid=(B_TOTAL // _MM_ROWS,),
        in_specs=[
            pl.BlockSpec((PROJ_DIM, _MM_ROWS),
                         lambda i: (jnp.int32(0), i)),
            pl.BlockSpec((PROJ_DIM, MODEL_DIM),
                         lambda i: (jnp.int32(0), jnp.int32(0))),
        ],
        out_specs=pl.BlockSpec((_MM_ROWS, MODEL_DIM),
                               lambda i: (i, jnp.int32(0))),
        out_shape=jax.ShapeDtypeStruct((B_TOTAL, MODEL_DIM), jnp.float32),
    )(embt, wt)


def kernel(ids, table, W):
    batch, seq = ids.shape
    ids32 = ids.astype(jnp.int32)
    ids_ext = jnp.concatenate(
        [jnp.zeros((8,), jnp.int32), ids32.reshape(-1)])
    tablet = table.astype(jnp.float32).T
    tail = jnp.pad(tablet[:, TAIL2_OFF:],
                   ((0, 0), (0, TAIL2 - (NUM_BUCKETS - TAIL2_OFF))))
    embt = _sc_gather(ids_ext, tablet, tail.reshape(-1))
    out = _project(embt, W.astype(jnp.float32).T)
    return out.reshape(batch, seq, MODEL_DIM)


# speedup vs baseline: 1.0404x; 1.0404x over previous
import functools

import jax
import jax.numpy as jnp
from jax import lax
from jax.experimental import pallas as pl
from jax.experimental.pallas import tpu as pltpu
from jax.experimental.pallas import tpu_sc as plsc

NUM_BUCKETS = 1000000
PROJ_DIM = 64
MODEL_DIM = 768
LANES = 16

NUM_CORES = 2
NUM_SUBCORES = 16
D_PER_CORE = PROJ_DIM // NUM_CORES

B_TOTAL = 4 * 8192
TOK_PER_TEC = B_TOTAL // NUM_SUBCORES
GCHUNK = 128
N_GC = TOK_PER_TEC // GCHUNK


def _sc_body(idse_hbm, tt_hbm, embt_hbm,
             row_sh0, row_sh1, ids_v, idx_v, vals_v, sem_f, sem_g):
    cid = lax.axis_index("c")
    sid = lax.axis_index("s")
    d0 = cid * D_PER_CORE
    base = sid * TOK_PER_TEC

    @pl.when(sid == jnp.int32(0))
    def _prime_start():
        pltpu.async_copy(tt_hbm.at[d0], row_sh0, sem_f)

    pltpu.sync_copy(idse_hbm.at[pl.ds(base, TOK_PER_TEC + 8)], ids_v)

    def _hash_step(_, carry):
        off, jrow, icol = carry
        p = ids_v[pl.ds(off + 7, LANES)]
        a = ids_v[pl.ds(off + 8, LANES)]
        t = ((p * 1234) % 1000) * 1000 + p * 567 + a
        idx_v[jrow, pl.ds(icol, LANES)] = t % NUM_BUCKETS
        nicol = icol + LANES
        wrap = nicol == GCHUNK
        return (off + LANES,
                jnp.where(wrap, jrow + 1, jrow),
                jnp.where(wrap, jnp.int32(0), nicol))

    lax.fori_loop(jnp.int32(0), jnp.int32(TOK_PER_TEC // LANES), _hash_step,
                  (jnp.int32(0), jnp.int32(0), jnp.int32(0)))

    @pl.when(jnp.logical_and(sid != jnp.int32(0),
                             sid % jnp.int32(4) == jnp.int32(0)))
    def _fix_row_start():
        v = idx_v[jnp.int32(0), pl.ds(0, LANES)]
        a0 = ids_v[pl.ds(8, LANES)]
        lane = lax.iota(jnp.int32, LANES)
        idx_v[jnp.int32(0), pl.ds(0, LANES)] = jnp.where(
            lane == 0, a0 % NUM_BUCKETS, v)

    @pl.when(sid == jnp.int32(0))
    def _prime_wait():
        pltpu.make_async_copy(tt_hbm.at[d0], row_sh0, sem_f).wait()

    plsc.subcore_barrier()

    bufs = (row_sh0, row_sh1)
    for d in range(D_PER_CORE):
        cur = bufs[d % 2]
        nxt = bufs[(d + 1) % 2]
        if d + 1 < D_PER_CORE:
            @pl.when(sid == jnp.int32(0))
            def _fill():
                pltpu.async_copy(tt_hbm.at[d0 + jnp.int32(d + 1)],
                                 nxt, sem_f)

        for h in range(2):
            copies = [
                pltpu.async_copy(
                    cur.at[idx_v.at[jnp.int32(h * N_GC // 2 + g)]],
                    vals_v.at[pl.ds(g * GCHUNK, GCHUNK)], sem_g)
                for g in range(N_GC // 2)
            ]
            for c in copies:
                c.wait()
            pltpu.sync_copy(
                vals_v,
                embt_hbm.at[d0 + jnp.int32(d)].at[
                    pl.ds(base + h * (TOK_PER_TEC // 2), TOK_PER_TEC // 2)])

        if d + 1 < D_PER_CORE:
            @pl.when(sid == jnp.int32(0))
            def _drain():
                pltpu.make_async_copy(tt_hbm.at[d0 + jnp.int32(d + 1)],
                                      nxt, sem_f).wait()

        plsc.subcore_barrier()


_sc_gather = functools.partial(
    pl.kernel,
    out_type=jax.ShapeDtypeStruct((PROJ_DIM, B_TOTAL), jnp.float32),
    mesh=plsc.VectorSubcoreMesh(core_axis_name="c", subcore_axis_name="s"),
    scratch_types=[
        pltpu.VMEM_SHARED((NUM_BUCKETS,), jnp.float32),
        pltpu.VMEM_SHARED((NUM_BUCKETS,), jnp.float32),
        pltpu.VMEM((TOK_PER_TEC + 8,), jnp.int32),
        pltpu.VMEM((N_GC, GCHUNK), jnp.int32),
        pltpu.VMEM((TOK_PER_TEC // 2,), jnp.float32),
        pltpu.SemaphoreType.DMA,
        pltpu.SemaphoreType.DMA,
    ],
)(_sc_body)


def _mm_body(embt_ref, wt_ref, out_ref):
    out_ref[...] = lax.dot_general(
        embt_ref[...], wt_ref[...],
        dimension_numbers=(((0,), (0,)), ((), ())),
        preferred_element_type=jnp.float32)


_MM_ROWS = 2048


def _project(embt, wt):
    return pl.pallas_call(
        _mm_body,
        grid=(B_TOTAL // _MM_ROWS,),
        in_specs=[
            pl.BlockSpec((PROJ_DIM, _MM_ROWS),
                         lambda i: (jnp.int32(0), i)),
            pl.BlockSpec((PROJ_DIM, MODEL_DIM),
                         lambda i: (jnp.int32(0), jnp.int32(0))),
        ],
        out_specs=pl.BlockSpec((_MM_ROWS, MODEL_DIM),
                               lambda i: (i, jnp.int32(0))),
        out_shape=jax.ShapeDtypeStruct((B_TOTAL, MODEL_DIM), jnp.float32),
    )(embt, wt)


def kernel(ids, table, W):
    batch, seq = ids.shape
    ids32 = ids.astype(jnp.int32)
    ids_ext = jnp.concatenate(
        [jnp.zeros((8,), jnp.int32), ids32.reshape(-1)])
    embt = _sc_gather(ids_ext, table.astype(jnp.float32).T)
    out = _project(embt, W.astype(jnp.float32).T)
    return out.reshape(batch, seq, MODEL_DIM)
